# baseline (speedup 1.0000x reference)
import jax
import jax.numpy as jnp
from jax import lax
from jax.experimental import pallas as pl
from jax.experimental.pallas import tpu as pltpu
from jax.experimental.pallas import tpu_sc as plsc

NUM_EMB = 1000
D = 64
N = 819200

NC = 2
NS = 16
NW = NC * NS
LANES = 16

TOKENS_PER_WORKER = N // NW
CHUNK = 128
STEPS = TOKENS_PER_WORKER // CHUNK
SLICES_PER_ROW = D // LANES


def _body(xt_hbm, idx_hbm, w_hbm, out_hbm, w_sh, idx_all,
          acc, rows, sem_g, sem_x, sem_out):
    cid = lax.axis_index("c")
    sid = lax.axis_index("s")
    wid = sid * NC + cid
    base0 = wid * TOKENS_PER_WORKER

    @pl.when(sid == 0)
    def _():
        pltpu.sync_copy(w_hbm, w_sh)

    pltpu.sync_copy(idx_hbm.at[pl.ds(base0, TOKENS_PER_WORKER)], idx_all)

    plsc.subcore_barrier()

    lane = lax.iota(jnp.int32, LANES)
    row_idx = [lane + c * LANES for c in range(SLICES_PER_ROW)]

    def idx_of(k):
        return idx_all.at[pl.ds(k * CHUNK, CHUNK)]

    def gather(k, b):
        return pltpu.make_async_copy(w_sh.at[idx_of(k)], rows[b], sem_g[b])

    def x_in(k, b):
        return pltpu.make_async_copy(
            xt_hbm.at[:, pl.ds(base0 + k * CHUNK, CHUNK)], acc[b], sem_x[b])

    def out_cp(k, b):
        return pltpu.make_async_copy(
            acc[b], out_hbm.at[:, pl.ds(base0 + k * CHUNK, CHUNK)],
            sem_out[b])

    gather(0, 0).start()
    x_in(0, 0).start()

    def pair(g, carry):
        for b in (0, 1):
            k = 2 * g + b
            b1 = 1 - b

            @pl.when(k + 1 < STEPS)
            def _():
                gather(k + 1, b1).start()

            @pl.when((k + 1 < STEPS) & (k >= 1))
            def _():
                out_cp(k - 1, b1).wait()

            @pl.when(k + 1 < STEPS)
            def _():
                x_in(k + 1, b1).start()

            gather(k, b).wait()
            x_in(k, b).wait()

            @plsc.parallel_loop(0, CHUNK, 1, unroll=4)
            def add_row(r):
                col = jnp.full((LANES,), r, jnp.int32)
                for c in range(SLICES_PER_ROW):
                    v = rows[b][r, pl.ds(c * LANES, LANES)]
                    plsc.addupdate_scatter(acc[b], [row_idx[c], col], v)

            out_cp(k, b).start()
        return carry

    lax.fori_loop(0, STEPS // 2, pair, 0)
    out_cp(STEPS - 2, 0).wait()
    out_cp(STEPS - 1, 1).wait()


@jax.jit
def _run(xt, idx, w):
    mesh = plsc.VectorSubcoreMesh(core_axis_name="c", subcore_axis_name="s")
    f = pl.kernel(
        _body,
        out_type=jax.ShapeDtypeStruct((D, N), jnp.float32),
        mesh=mesh,
        compiler_params=pltpu.CompilerParams(needs_layout_passes=False),
        scratch_types=[
            pltpu.VMEM_SHARED((NUM_EMB, 128), jnp.float32),
            pltpu.VMEM((TOKENS_PER_WORKER,), jnp.int32),
            [pltpu.VMEM((D, CHUNK), jnp.float32)] * 2,
            [pltpu.VMEM((CHUNK, 128), jnp.float32)] * 2,
            [pltpu.SemaphoreType.DMA] * 2,
            [pltpu.SemaphoreType.DMA] * 2,
            [pltpu.SemaphoreType.DMA] * 2,
        ],
    )
    return f(xt, idx, w)


def kernel(x, resids_positional_encoded, W):
    idx = resids_positional_encoded.astype(jnp.int32)
    w128 = jnp.pad(W, ((0, 0), (0, 128 - D)))
    return _run(x.T, idx, w128).T


# speedup vs baseline: 3.0283x; 3.0283x over previous
import jax
import jax.numpy as jnp
from jax import lax
from jax.experimental import pallas as pl
from jax.experimental.pallas import tpu as pltpu
from jax.experimental.pallas import tpu_sc as plsc

NUM_EMB = 1000
D = 64
N = 819200

NC = 2
NS = 16
NW = NC * NS
LANES = 16

TOKENS_PER_WORKER = N // NW
CHUNK = 128
STEPS = TOKENS_PER_WORKER // CHUNK
GROUPS = CHUNK // LANES


def _body(xt_hbm, idx_hbm, wt_hbm, out_hbm, w_v, idx_all, acc,
          sem_x, sem_out):
    cid = lax.axis_index("c")
    sid = lax.axis_index("s")
    wid = sid * NC + cid
    base0 = wid * TOKENS_PER_WORKER

    pltpu.sync_copy(wt_hbm, w_v)
    pltpu.sync_copy(idx_hbm.at[pl.ds(base0, TOKENS_PER_WORKER)], idx_all)

    def x_in(k, b):
        return pltpu.make_async_copy(
            xt_hbm.at[:, pl.ds(base0 + k * CHUNK, CHUNK)], acc[b], sem_x[b])

    def out_cp(k, b):
        return pltpu.make_async_copy(
            acc[b], out_hbm.at[:, pl.ds(base0 + k * CHUNK, CHUNK)],
            sem_out[b])

    x_in(0, 0).start()

    def pair(g, carry):
        for b in (0, 1):
            k = 2 * g + b
            b1 = 1 - b

            @pl.when((k + 1 < STEPS) & (k >= 1))
            def _():
                out_cp(k - 1, b1).wait()

            @pl.when(k + 1 < STEPS)
            def _():
                x_in(k + 1, b1).start()

            x_in(k, b).wait()

            @plsc.parallel_loop(0, GROUPS, 1)
            def add_group(gg):
                ids = idx_all[pl.ds(k * CHUNK + gg * LANES, LANES)]
                for f in range(D):
                    wv = plsc.load_gather(w_v, [ids + f * NUM_EMB])
                    plsc.addupdate(acc[b].at[f, pl.ds(gg * LANES, LANES)],
                                   wv)

            out_cp(k, b).start()
        return carry

    lax.fori_loop(0, STEPS // 2, pair, 0)
    out_cp(STEPS - 2, 0).wait()
    out_cp(STEPS - 1, 1).wait()


@jax.jit
def _run(xt, idx, wt):
    mesh = plsc.VectorSubcoreMesh(core_axis_name="c", subcore_axis_name="s")
    f = pl.kernel(
        _body,
        out_type=jax.ShapeDtypeStruct((D, N), jnp.float32),
        mesh=mesh,
        compiler_params=pltpu.CompilerParams(needs_layout_passes=False),
        scratch_types=[
            pltpu.VMEM((NUM_EMB * D,), jnp.float32),
            pltpu.VMEM((TOKENS_PER_WORKER,), jnp.int32),
            [pltpu.VMEM((D, CHUNK), jnp.float32)] * 2,
            [pltpu.SemaphoreType.DMA] * 2,
            [pltpu.SemaphoreType.DMA] * 2,
        ],
    )
    return f(xt, idx, wt)


def kernel(x, resids_positional_encoded, W):
    idx = resids_positional_encoded.astype(jnp.int32)
    wt = jnp.reshape(W.T, (-1,))
    return _run(x.T, idx, wt).T


# speedup vs baseline: 3.8457x; 1.2699x over previous
import jax
import jax.numpy as jnp
from jax import lax
from jax.experimental import pallas as pl
from jax.experimental.pallas import tpu as pltpu
from jax.experimental.pallas import tpu_sc as plsc

NUM_EMB = 1000
D = 64
N = 819200

NC = 2
NS = 16
NW = NC * NS
LANES = 16

TOKENS_PER_WORKER = N // NW
CHUNK = 256
STEPS = TOKENS_PER_WORKER // CHUNK
GROUPS = CHUNK // LANES


def _body(xt_hbm, idx_hbm, wt_hbm, out_hbm, w_v, idx_all, acc,
          sem_x, sem_out):
    cid = lax.axis_index("c")
    sid = lax.axis_index("s")
    wid = sid * NC + cid
    base0 = wid * TOKENS_PER_WORKER

    pltpu.sync_copy(wt_hbm, w_v)
    pltpu.sync_copy(idx_hbm.at[pl.ds(base0, TOKENS_PER_WORKER)], idx_all)

    def x_in(k, b):
        return pltpu.make_async_copy(
            xt_hbm.at[:, pl.ds(base0 + k * CHUNK, CHUNK)], acc[b], sem_x[b])

    def out_cp(k, b):
        return pltpu.make_async_copy(
            acc[b], out_hbm.at[:, pl.ds(base0 + k * CHUNK, CHUNK)],
            sem_out[b])

    x_in(0, 0).start()

    def pair(g, carry):
        for b in (0, 1):
            k = 2 * g + b
            b1 = 1 - b

            @pl.when((k + 1 < STEPS) & (k >= 1))
            def _():
                out_cp(k - 1, b1).wait()

            @pl.when(k + 1 < STEPS)
            def _():
                x_in(k + 1, b1).start()

            x_in(k, b).wait()

            @plsc.parallel_loop(0, GROUPS, 1)
            def add_group(gg):
                ids = idx_all[pl.ds(k * CHUNK + gg * LANES, LANES)]
                for f in range(D):
                    wv = plsc.load_gather(w_v, [ids + f * NUM_EMB])
                    plsc.addupdate(acc[b].at[f, pl.ds(gg * LANES, LANES)],
                                   wv)

            out_cp(k, b).start()
        return carry

    lax.fori_loop(0, STEPS // 2, pair, 0)
    out_cp(STEPS - 2, 0).wait()
    out_cp(STEPS - 1, 1).wait()


@jax.jit
def _run(xt, idx, wt):
    mesh = plsc.VectorSubcoreMesh(core_axis_name="c", subcore_axis_name="s")
    f = pl.kernel(
        _body,
        out_type=jax.ShapeDtypeStruct((D, N), jnp.float32),
        mesh=mesh,
        compiler_params=pltpu.CompilerParams(needs_layout_passes=False),
        scratch_types=[
            pltpu.VMEM((NUM_EMB * D,), jnp.float32),
            pltpu.VMEM((TOKENS_PER_WORKER,), jnp.int32),
            [pltpu.VMEM((D, CHUNK), jnp.float32)] * 2,
            [pltpu.SemaphoreType.DMA] * 2,
            [pltpu.SemaphoreType.DMA] * 2,
        ],
    )
    return f(xt, idx, wt)


def kernel(x, resids_positional_encoded, W):
    idx = resids_positional_encoded.astype(jnp.int32)
    wt = jnp.reshape(W.T, (-1,))
    return _run(x.T, idx, wt).T


# speedup vs baseline: 4.4575x; 1.1591x over previous
import jax
import jax.numpy as jnp
from jax import lax
from jax.experimental import pallas as pl
from jax.experimental.pallas import tpu as pltpu
from jax.experimental.pallas import tpu_sc as plsc

NUM_EMB = 1000
D = 64
N = 819200

NC = 2
NS = 16
NW = NC * NS
LANES = 16

TOKENS_PER_WORKER = N // NW
CHUNK = 512
STEPS = TOKENS_PER_WORKER // CHUNK
GROUPS = CHUNK // LANES


def _body(xt_hbm, idx_hbm, wt_hbm, out_hbm, w_v, acc, idxb,
          sem_x, sem_out):
    cid = lax.axis_index("c")
    sid = lax.axis_index("s")
    wid = sid * NC + cid
    base0 = wid * TOKENS_PER_WORKER

    pltpu.sync_copy(wt_hbm, w_v)

    def x_in(k, b):
        return pltpu.make_async_copy(
            xt_hbm.at[:, pl.ds(base0 + k * CHUNK, CHUNK)], acc[b], sem_x[b])

    def i_in(k, b):
        return pltpu.make_async_copy(
            idx_hbm.at[pl.ds(base0 + k * CHUNK, CHUNK)], idxb[b], sem_x[b])

    def out_cp(k, b):
        return pltpu.make_async_copy(
            acc[b], out_hbm.at[:, pl.ds(base0 + k * CHUNK, CHUNK)],
            sem_out[b])

    x_in(0, 0).start()
    i_in(0, 0).start()

    def pair(g, carry):
        for b in (0, 1):
            k = 2 * g + b
            b1 = 1 - b

            @pl.when((k + 1 < STEPS) & (k >= 1))
            def _():
                out_cp(k - 1, b1).wait()

            @pl.when(k + 1 < STEPS)
            def _():
                x_in(k + 1, b1).start()
                i_in(k + 1, b1).start()

            x_in(k, b).wait()
            i_in(k, b).wait()

            @plsc.parallel_loop(0, GROUPS, 1)
            def add_group(gg):
                ids = idxb[b][pl.ds(gg * LANES, LANES)]
                for f in range(D):
                    wv = plsc.load_gather(w_v, [ids + f * NUM_EMB])
                    plsc.addupdate(acc[b].at[f, pl.ds(gg * LANES, LANES)],
                                   wv)

            out_cp(k, b).start()
        return carry

    lax.fori_loop(0, STEPS // 2, pair, 0)
    out_cp(STEPS - 2, 0).wait()
    out_cp(STEPS - 1, 1).wait()


@jax.jit
def _run(xt, idx, wt):
    mesh = plsc.VectorSubcoreMesh(core_axis_name="c", subcore_axis_name="s")
    f = pl.kernel(
        _body,
        out_type=jax.ShapeDtypeStruct((D, N), jnp.float32),
        mesh=mesh,
        compiler_params=pltpu.CompilerParams(needs_layout_passes=False),
        scratch_types=[
            pltpu.VMEM((NUM_EMB * D,), jnp.float32),
            [pltpu.VMEM((D, CHUNK), jnp.float32)] * 2,
            [pltpu.VMEM((CHUNK,), jnp.int32)] * 2,
            [pltpu.SemaphoreType.DMA] * 2,
            [pltpu.SemaphoreType.DMA] * 2,
        ],
    )
    return f(xt, idx, wt)


def kernel(x, resids_positional_encoded, W):
    idx = resids_positional_encoded.astype(jnp.int32)
    wt = jnp.reshape(W.T, (-1,))
    return _run(x.T, idx, wt).T


# speedup vs baseline: 5.4123x; 1.2142x over previous
import jax
import jax.numpy as jnp
from jax import lax
from jax.experimental import pallas as pl
from jax.experimental.pallas import tpu as pltpu
from jax.experimental.pallas import tpu_sc as plsc

NUM_EMB = 1000
D = 64
N = 819200

NC = 2
NS = 16
NW = NC * NS
LANES = 16

FEATS = 8
NQ = NW // (D // FEATS)
TOKENS_PER_Q = N // NQ
CHUNK = 2048
STEPS = TOKENS_PER_Q // CHUNK
GROUPS = CHUNK // LANES
NBUF = 4


def _body(xt_hbm, idx_hbm, wt_hbm, out_hbm, w_v, acc, idxb,
          sem_x, sem_out):
    cid = lax.axis_index("c")
    sid = lax.axis_index("s")
    wid = sid * NC + cid
    octet = wid % (D // FEATS)
    quarter = wid // (D // FEATS)
    f0 = octet * FEATS
    qbase = quarter * TOKENS_PER_Q

    pltpu.sync_copy(wt_hbm.at[pl.ds(f0 * NUM_EMB, FEATS * NUM_EMB)], w_v)

    def x_in(k, b):
        return pltpu.make_async_copy(
            xt_hbm.at[pl.ds(f0, FEATS), pl.ds(qbase + k * CHUNK, CHUNK)],
            acc[b], sem_x[b])

    def i_in(k, b):
        return pltpu.make_async_copy(
            idx_hbm.at[pl.ds(qbase + k * CHUNK, CHUNK)], idxb[b], sem_x[b])

    def out_cp(k, b):
        return pltpu.make_async_copy(
            acc[b],
            out_hbm.at[pl.ds(f0, FEATS), pl.ds(qbase + k * CHUNK, CHUNK)],
            sem_out[b])

    for kk in (0, 1):
        x_in(kk, kk).start()
        i_in(kk, kk).start()

    def quad(g, carry):
        for j in range(NBUF):
            k = NBUF * g + j
            b = j

            x_in(k, b).wait()
            i_in(k, b).wait()

            @plsc.parallel_loop(0, GROUPS, 1)
            def add_group(gg):
                ids = idxb[b][pl.ds(gg * LANES, LANES)]
                for f in range(FEATS):
                    wv = plsc.load_gather(w_v, [ids + f * NUM_EMB])
                    plsc.addupdate(acc[b].at[f, pl.ds(gg * LANES, LANES)],
                                   wv)

            out_cp(k, b).start()

            b2 = (j + 2) % NBUF
            @pl.when(k + 2 < STEPS)
            def _():
                @pl.when(k >= 2)
                def _():
                    out_cp(k - 2, b2).wait()
                x_in(k + 2, b2).start()
                i_in(k + 2, b2).start()

        return carry

    lax.fori_loop(0, STEPS // NBUF, quad, 0)
    for k in range(STEPS - NBUF, STEPS):
        out_cp(k, k % NBUF).wait()


@jax.jit
def _run(xt, idx, wt):
    mesh = plsc.VectorSubcoreMesh(core_axis_name="c", subcore_axis_name="s")
    f = pl.kernel(
        _body,
        out_type=jax.ShapeDtypeStruct((D, N), jnp.float32),
        mesh=mesh,
        compiler_params=pltpu.CompilerParams(needs_layout_passes=False),
        scratch_types=[
            pltpu.VMEM((FEATS * NUM_EMB,), jnp.float32),
            [pltpu.VMEM((FEATS, CHUNK), jnp.float32)] * NBUF,
            [pltpu.VMEM((CHUNK,), jnp.int32)] * NBUF,
            [pltpu.SemaphoreType.DMA] * NBUF,
            [pltpu.SemaphoreType.DMA] * NBUF,
        ],
    )
    return f(xt, idx, wt)


def kernel(x, resids_positional_encoded, W):
    idx = resids_positional_encoded.astype(jnp.int32)
    wt = jnp.reshape(W.T, (-1,))
    return _run(x.T, idx, wt).T


# speedup vs baseline: 5.6107x; 1.0367x over previous
import jax
import jax.numpy as jnp
from jax import lax
from jax.experimental import pallas as pl
from jax.experimental.pallas import tpu as pltpu
from jax.experimental.pallas import tpu_sc as plsc

NUM_EMB = 1000
D = 64
N = 819200

NC = 2
NS = 16
NW = NC * NS
LANES = 16

FEATS = 16
NQ = NW // (D // FEATS)
TOKENS_PER_Q = N // NQ
CHUNK = 1024
STEPS = TOKENS_PER_Q // CHUNK
GROUPS = CHUNK // LANES
NBUF = 4


def _body(xt_hbm, idx_hbm, wt_hbm, out_hbm, w_v, acc, idxb,
          sem_x, sem_out):
    cid = lax.axis_index("c")
    sid = lax.axis_index("s")
    wid = sid * NC + cid
    octet = wid % (D // FEATS)
    quarter = wid // (D // FEATS)
    f0 = octet * FEATS
    qbase = quarter * TOKENS_PER_Q

    pltpu.sync_copy(wt_hbm.at[pl.ds(f0 * NUM_EMB, FEATS * NUM_EMB)], w_v)

    def x_in(k, b):
        return pltpu.make_async_copy(
            xt_hbm.at[pl.ds(f0, FEATS), pl.ds(qbase + k * CHUNK, CHUNK)],
            acc[b], sem_x[b])

    def i_in(k, b):
        return pltpu.make_async_copy(
            idx_hbm.at[pl.ds(qbase + k * CHUNK, CHUNK)], idxb[b], sem_x[b])

    def out_cp(k, b):
        return pltpu.make_async_copy(
            acc[b],
            out_hbm.at[pl.ds(f0, FEATS), pl.ds(qbase + k * CHUNK, CHUNK)],
            sem_out[b])

    for kk in (0, 1):
        x_in(kk, kk).start()
        i_in(kk, kk).start()

    def quad(g, carry):
        for j in range(NBUF):
            k = NBUF * g + j
            b = j

            x_in(k, b).wait()
            i_in(k, b).wait()

            @plsc.parallel_loop(0, GROUPS, 1)
            def add_group(gg):
                ids = idxb[b][pl.ds(gg * LANES, LANES)]
                for f in range(FEATS):
                    wv = plsc.load_gather(w_v, [ids + f * NUM_EMB])
                    plsc.addupdate(acc[b].at[f, pl.ds(gg * LANES, LANES)],
                                   wv)

            out_cp(k, b).start()

            b2 = (j + 2) % NBUF
            @pl.when(k + 2 < STEPS)
            def _():
                @pl.when(k >= 2)
                def _():
                    out_cp(k - 2, b2).wait()
                x_in(k + 2, b2).start()
                i_in(k + 2, b2).start()

        return carry

    lax.fori_loop(0, STEPS // NBUF, quad, 0)
    for k in range(STEPS - NBUF, STEPS):
        out_cp(k, k % NBUF).wait()


@jax.jit
def _run(xt, idx, wt):
    mesh = plsc.VectorSubcoreMesh(core_axis_name="c", subcore_axis_name="s")
    f = pl.kernel(
        _body,
        out_type=jax.ShapeDtypeStruct((D, N), jnp.float32),
        mesh=mesh,
        compiler_params=pltpu.CompilerParams(needs_layout_passes=False),
        scratch_types=[
            pltpu.VMEM((FEATS * NUM_EMB,), jnp.float32),
            [pltpu.VMEM((FEATS, CHUNK), jnp.float32)] * NBUF,
            [pltpu.VMEM((CHUNK,), jnp.int32)] * NBUF,
            [pltpu.SemaphoreType.DMA] * NBUF,
            [pltpu.SemaphoreType.DMA] * NBUF,
        ],
    )
    return f(xt, idx, wt)


def kernel(x, resids_positional_encoded, W):
    idx = resids_positional_encoded.astype(jnp.int32)
    wt = jnp.reshape(W.T, (-1,))
    return _run(x.T, idx, wt).T


# speedup vs baseline: 5.8585x; 1.0442x over previous
import jax
import jax.numpy as jnp
from jax import lax
from jax.experimental import pallas as pl
from jax.experimental.pallas import tpu as pltpu
from jax.experimental.pallas import tpu_sc as plsc

NUM_EMB = 1000
D = 64
N = 819200

NC = 2
NS = 16
NW = NC * NS
LANES = 16

FEATS = 16
NQ = NW // (D // FEATS)
TOKENS_PER_Q = N // NQ
CHUNK = 1280
STEPS = TOKENS_PER_Q // CHUNK
GROUPS = CHUNK // LANES
NBUF = 4


def _body(xt_hbm, idx_hbm, wt_hbm, out_hbm, w_v, acc, idxb,
          sem_x, sem_out):
    cid = lax.axis_index("c")
    sid = lax.axis_index("s")
    wid = sid * NC + cid
    octet = wid % (D // FEATS)
    quarter = wid // (D // FEATS)
    f0 = octet * FEATS
    qbase = quarter * TOKENS_PER_Q

    pltpu.sync_copy(wt_hbm.at[pl.ds(f0 * NUM_EMB, FEATS * NUM_EMB)], w_v)

    def x_in(k, b):
        return pltpu.make_async_copy(
            xt_hbm.at[pl.ds(f0, FEATS), pl.ds(qbase + k * CHUNK, CHUNK)],
            acc[b], sem_x[b])

    def i_in(k, b):
        return pltpu.make_async_copy(
            idx_hbm.at[pl.ds(qbase + k * CHUNK, CHUNK)], idxb[b], sem_x[b])

    def out_cp(k, b):
        return pltpu.make_async_copy(
            acc[b],
            out_hbm.at[pl.ds(f0, FEATS), pl.ds(qbase + k * CHUNK, CHUNK)],
            sem_out[b])

    for kk in (0, 1):
        x_in(kk, kk).start()
        i_in(kk, kk).start()

    def quad(g, carry):
        for j in range(NBUF):
            k = NBUF * g + j
            b = j

            x_in(k, b).wait()
            i_in(k, b).wait()

            @plsc.parallel_loop(0, GROUPS, 1)
            def add_group(gg):
                ids = idxb[b][pl.ds(gg * LANES, LANES)]
                for f in range(FEATS):
                    wv = plsc.load_gather(w_v, [ids + f * NUM_EMB])
                    plsc.addupdate(acc[b].at[f, pl.ds(gg * LANES, LANES)],
                                   wv)

            out_cp(k, b).start()

            b2 = (j + 2) % NBUF
            @pl.when(k + 2 < STEPS)
            def _():
                @pl.when(k >= 2)
                def _():
                    out_cp(k - 2, b2).wait()
                x_in(k + 2, b2).start()
                i_in(k + 2, b2).start()

        return carry

    lax.fori_loop(0, STEPS // NBUF, quad, 0)
    for k in range(STEPS - NBUF, STEPS):
        out_cp(k, k % NBUF).wait()


@jax.jit
def _run(xt, idx, wt):
    mesh = plsc.VectorSubcoreMesh(core_axis_name="c", subcore_axis_name="s")
    f = pl.kernel(
        _body,
        out_type=jax.ShapeDtypeStruct((D, N), jnp.float32),
        mesh=mesh,
        compiler_params=pltpu.CompilerParams(needs_layout_passes=False),
        scratch_types=[
            pltpu.VMEM((FEATS * NUM_EMB,), jnp.float32),
            [pltpu.VMEM((FEATS, CHUNK), jnp.float32)] * NBUF,
            [pltpu.VMEM((CHUNK,), jnp.int32)] * NBUF,
            [pltpu.SemaphoreType.DMA] * NBUF,
            [pltpu.SemaphoreType.DMA] * NBUF,
        ],
    )
    return f(xt, idx, wt)


def kernel(x, resids_positional_encoded, W):
    idx = resids_positional_encoded.astype(jnp.int32)
    wt = jnp.reshape(W.T, (-1,))
    return _run(x.T, idx, wt).T


# speedup vs baseline: 6.1230x; 1.0451x over previous
import jax
import jax.numpy as jnp
from jax import lax
from jax.experimental import pallas as pl
from jax.experimental.pallas import tpu as pltpu
from jax.experimental.pallas import tpu_sc as plsc

NUM_EMB = 1000
D = 64
N = 819200

NC = 2
NS = 16
NW = NC * NS
LANES = 16

FEATS = 16
NQ = NW // (D // FEATS)
TOKENS_PER_Q = N // NQ
CHUNK = 2048
STEPS = TOKENS_PER_Q // CHUNK
GROUPS = CHUNK // LANES
NBUF = 3
OUTER = (STEPS + NBUF - 1) // NBUF


def _body(xt_hbm, idx_hbm, wt_hbm, out_hbm, w_v, acc, idxb,
          sem_x, sem_out):
    cid = lax.axis_index("c")
    sid = lax.axis_index("s")
    wid = sid * NC + cid
    octet = wid % (D // FEATS)
    quarter = wid // (D // FEATS)
    f0 = octet * FEATS
    qbase = quarter * TOKENS_PER_Q

    pltpu.sync_copy(wt_hbm.at[pl.ds(f0 * NUM_EMB, FEATS * NUM_EMB)], w_v)

    def x_in(k, b):
        return pltpu.make_async_copy(
            xt_hbm.at[pl.ds(f0, FEATS), pl.ds(qbase + k * CHUNK, CHUNK)],
            acc[b], sem_x[b])

    def i_in(k, b):
        return pltpu.make_async_copy(
            idx_hbm.at[pl.ds(qbase + k * CHUNK, CHUNK)], idxb[b], sem_x[b])

    def out_cp(k, b):
        return pltpu.make_async_copy(
            acc[b],
            out_hbm.at[pl.ds(f0, FEATS), pl.ds(qbase + k * CHUNK, CHUNK)],
            sem_out[b])

    for kk in (0, 1):
        x_in(kk, kk).start()
        i_in(kk, kk).start()

    def round_(g, carry):
        for j in range(NBUF):
            k = NBUF * g + j
            b = j

            @pl.when(k < STEPS)
            def _():
                x_in(k, b).wait()
                i_in(k, b).wait()

                @plsc.parallel_loop(0, GROUPS, 1)
                def add_group(gg):
                    ids = idxb[b][pl.ds(gg * LANES, LANES)]
                    for f in range(FEATS):
                        wv = plsc.load_gather(w_v, [ids + f * NUM_EMB])
                        plsc.addupdate(
                            acc[b].at[f, pl.ds(gg * LANES, LANES)], wv)

                out_cp(k, b).start()

                b2 = (j + 2) % NBUF

                @pl.when(k + 2 < STEPS)
                def _():
                    @pl.when(k >= 1)
                    def _():
                        out_cp(k - 1, b2).wait()
                    x_in(k + 2, b2).start()
                    i_in(k + 2, b2).start()

        return carry

    lax.fori_loop(0, OUTER, round_, 0)
    for k in range(STEPS - NBUF, STEPS):
        out_cp(k, k % NBUF).wait()


@jax.jit
def _run(xt, idx, wt):
    mesh = plsc.VectorSubcoreMesh(core_axis_name="c", subcore_axis_name="s")
    f = pl.kernel(
        _body,
        out_type=jax.ShapeDtypeStruct((D, N), jnp.float32),
        mesh=mesh,
        compiler_params=pltpu.CompilerParams(needs_layout_passes=False),
        scratch_types=[
            pltpu.VMEM((FEATS * NUM_EMB,), jnp.float32),
            [pltpu.VMEM((FEATS, CHUNK), jnp.float32)] * NBUF,
            [pltpu.VMEM((CHUNK,), jnp.int32)] * NBUF,
            [pltpu.SemaphoreType.DMA] * NBUF,
            [pltpu.SemaphoreType.DMA] * NBUF,
        ],
    )
    return f(xt, idx, wt)


def kernel(x, resids_positional_encoded, W):
    idx = resids_positional_encoded.astype(jnp.int32)
    wt = jnp.reshape(W.T, (-1,))
    return _run(x.T, idx, wt).T


# speedup vs baseline: 6.1473x; 1.0040x over previous
import jax
import jax.numpy as jnp
from jax import lax
from jax.experimental import pallas as pl
from jax.experimental.pallas import tpu as pltpu
from jax.experimental.pallas import tpu_sc as plsc

NUM_EMB = 1000
D = 64
N = 819200

NC = 2
NS = 16
NW = NC * NS
LANES = 16

FEATS = 16
NQ = NW // (D // FEATS)
TOKENS_PER_Q = N // NQ
CHUNK = 2048
STEPS = TOKENS_PER_Q // CHUNK
GROUPS = CHUNK // LANES
NBUF = 3
OUTER = (STEPS + NBUF - 1) // NBUF


def _body(xt_hbm, idx_hbm, wt_hbm, out_hbm, w_v, acc, idxb,
          sem_x, sem_out):
    cid = lax.axis_index("c")
    sid = lax.axis_index("s")
    wid = sid * NC + cid
    fgroup = wid % (D // FEATS)
    shard = wid // (D // FEATS)
    f0 = fgroup * FEATS
    qbase = shard * TOKENS_PER_Q

    pltpu.sync_copy(wt_hbm.at[pl.ds(f0 * NUM_EMB, FEATS * NUM_EMB)], w_v)

    def x_in(k, b):
        return pltpu.make_async_copy(
            xt_hbm.at[pl.ds(f0, FEATS), pl.ds(qbase + k * CHUNK, CHUNK)],
            acc[b], sem_x[b])

    def i_in(k, b):
        return pltpu.make_async_copy(
            idx_hbm.at[pl.ds(qbase + k * CHUNK, CHUNK)], idxb[b], sem_x[b])

    def out_cp(k, b):
        return pltpu.make_async_copy(
            acc[b],
            out_hbm.at[pl.ds(f0, FEATS), pl.ds(qbase + k * CHUNK, CHUNK)],
            sem_out[b])

    for kk in (0, 1):
        x_in(kk, kk).start()
        i_in(kk, kk).start()

    def round_(g, carry):
        for j in range(NBUF):
            k = NBUF * g + j
            b = j

            @pl.when(k < STEPS)
            def _():
                x_in(k, b).wait()
                i_in(k, b).wait()

                @plsc.parallel_loop(0, GROUPS, 1)
                def add_group(gg):
                    ids = idxb[b][pl.ds(gg * LANES, LANES)]
                    for f in range(FEATS):
                        wv = plsc.load_gather(w_v, [ids + f * NUM_EMB])
                        plsc.addupdate(
                            acc[b].at[f, pl.ds(gg * LANES, LANES)], wv)

                out_cp(k, b).start()

                b2 = (j + 2) % NBUF

                @pl.when(k + 2 < STEPS)
                def _():
                    @pl.when(k >= 1)
                    def _():
                        out_cp(k - 1, b2).wait()
                    x_in(k + 2, b2).start()
                    i_in(k + 2, b2).start()

        return carry

    lax.fori_loop(0, OUTER, round_, 0)
    for k in range(STEPS - NBUF, STEPS):
        out_cp(k, k % NBUF).wait()


@jax.jit
def _run(xt, idx, wt):
    mesh = plsc.VectorSubcoreMesh(core_axis_name="c", subcore_axis_name="s")
    f = pl.kernel(
        _body,
        out_type=jax.ShapeDtypeStruct((D, N), jnp.float32),
        mesh=mesh,
        compiler_params=pltpu.CompilerParams(needs_layout_passes=False),
        scratch_types=[
            pltpu.VMEM((FEATS * NUM_EMB,), jnp.float32),
            [pltpu.VMEM((FEATS, CHUNK), jnp.float32)] * NBUF,
            [pltpu.VMEM((CHUNK,), jnp.int32)] * NBUF,
            [pltpu.SemaphoreType.DMA] * NBUF,
            [pltpu.SemaphoreType.DMA] * NBUF,
        ],
    )
    return f(xt, idx, wt)


def kernel(x, resids_positional_encoded, W):
    idx = resids_positional_encoded.astype(jnp.int32)
    wt = jnp.reshape(W.T, (-1,))
    return _run(x.T, idx, wt).T
